# baseline (speedup 1.0000x reference)
import functools

import jax
import jax.numpy as jnp
import numpy as np
from jax import lax
from jax.experimental import pallas as pl
from jax.experimental.pallas import tpu as pltpu

_TF_ROTS = ((13, 15, 26, 6), (17, 29, 16, 24))
_TINY = np.float32(1.1754943508222875e-38)


def _threefry_bits(x1):
    ks = (jnp.uint32(0), jnp.uint32(42), jnp.uint32(42 ^ 0x1BD11BDA))
    x1 = x1 + ks[1]
    x0 = x1
    x1 = (x1 << 13) | (x1 >> 19)
    x1 = x0 ^ x1
    first = True
    for i in range(5):
        rots = _TF_ROTS[i % 2]
        for r in (rots[1:] if first else rots):
            x0 = x0 + x1
            x1 = (x1 << r) | (x1 >> (32 - r))
            x1 = x0 ^ x1
        first = False
        x0 = x0 + ks[(i + 1) % 3]
        x1 = x1 + ks[(i + 2) % 3] + jnp.uint32(i + 1)
    return x0 ^ x1


def _gumbel(bits):
    fb = (bits >> 9) | jnp.uint32(0x3F800000)
    f = lax.bitcast_convert_type(fb, jnp.float32) - jnp.float32(1.0)
    u = jnp.maximum(f, _TINY)
    return -jnp.log(-jnp.log(u))


def _body(x_ref, o_ref, accv_ref, accc_ref, *, S, R, CT, C, NC):
    b = pl.program_id(0)
    k = pl.program_id(1)
    NS = R // S

    lanes = lax.broadcasted_iota(jnp.int32, (S, 128), 1)
    rows_i = lax.broadcasted_iota(jnp.int32, (S, 128), 0)
    t = lanes & 15
    grp = lanes >> 4
    pre_c = rows_i * 8 + grp + k * (R * 8)
    pre_idx = b * CT + t * C + pre_c

    def make_step(masked):
        def step(i, carry):
            acc_v, acc_c = carry
            off = i * (S * 8)
            cv = pre_c + off
            bits = _threefry_bits((pre_idx + off).astype(jnp.uint32))
            g = _gumbel(bits)
            val = x_ref[0, pl.ds(i * S, S), :] + g
            if masked:
                val = jnp.where(cv < C, val, -jnp.inf)
            upd = val > acc_v
            acc_v = jnp.maximum(acc_v, val)
            acc_c = jnp.where(upd, cv, acc_c)
            return acc_v, acc_c
        return step

    @pl.when(k == 0)
    def _init():
        accv_ref[...] = jnp.full((S, 128), -jnp.inf, jnp.float32)
        accc_ref[...] = jnp.zeros((S, 128), jnp.int32)

    acc0 = (accv_ref[...], accc_ref[...])

    @pl.when(k < NC - 1)
    def _mid():
        v, c = lax.fori_loop(0, NS, make_step(False), acc0)
        accv_ref[...] = v
        accc_ref[...] = c

    @pl.when(k == NC - 1)
    def _last():
        bv, bc = lax.fori_loop(0, NS, make_step(True), acc0)
        n = S
        while n > 1:
            h = n // 2
            v1, v2 = bv[:h], bv[h:n]
            c1, c2 = bc[:h], bc[h:n]
            take = (v2 > v1) | ((v2 == v1) & (c2 < c1))
            bv = jnp.where(take, v2, v1)
            bc = jnp.where(take, c2, c1)
            n = h
        best_v = bv[:, 0:16]
        best_c = bc[:, 0:16]
        for gi in range(1, 8):
            vv = bv[:, gi * 16:(gi + 1) * 16]
            cc = bc[:, gi * 16:(gi + 1) * 16]
            take = (vv > best_v) | ((vv == best_v) & (cc < best_c))
            best_v = jnp.where(take, vv, best_v)
            best_c = jnp.where(take, cc, best_c)
        o_ref[0, 0, :] = best_c[0]


def _sampler(Xr, *, B, C, T, S, R, NC, interpret=False):
    return pl.pallas_call(
        functools.partial(_body, S=S, R=R, CT=C * T, C=C, NC=NC),
        grid=(B, NC),
        in_specs=[pl.BlockSpec((1, R, 128), lambda b, k: (b, k, 0))],
        out_specs=pl.BlockSpec((1, 1, 16), lambda b, k: (b, 0, 0)),
        out_shape=jax.ShapeDtypeStruct((B, 1, 16), jnp.int32),
        scratch_shapes=[pltpu.VMEM((S, 128), jnp.float32),
                        pltpu.VMEM((S, 128), jnp.int32)],
        compiler_params=pltpu.CompilerParams(
            dimension_semantics=("parallel", "arbitrary")),
        interpret=interpret,
    )(Xr)


def kernel(X, interpret=False):
    if X.ndim == 2:
        X = X[None]
    B, C, T = X.shape
    CT = C * T
    assert T == 16 and CT % 128 == 0, (B, C, T)
    ROWS = CT // 128
    S = 32
    R = min(512, ((ROWS + S - 1) // S) * S)
    NC = (ROWS + R - 1) // R
    Xr = X.reshape(B, ROWS, 128)
    out = _sampler(Xr, B=B, C=C, T=T, S=S, R=R, NC=NC, interpret=interpret)
    return out.reshape(B, T)


# speedup vs baseline: 2.5486x; 2.5486x over previous
import functools

import jax
import jax.numpy as jnp
import numpy as np
from jax import lax
from jax.experimental import pallas as pl
from jax.experimental.pallas import tpu as pltpu

_TF_ROTS = ((13, 15, 26, 6), (17, 29, 16, 24))
_TINY = np.float32(1.1754943508222875e-38)


def _threefry_bits(x1):
    k1 = np.uint32(42)
    k2 = np.uint32(42 ^ 0x1BD11BDA)
    ks = (np.uint32(0), k1, k2)
    inj = [(ks[(i + 1) % 3], np.uint32(ks[(i + 2) % 3] + np.uint32(i + 1)))
           for i in range(5)]
    x1 = x1 + k1
    x0 = x1
    x1 = (x1 << 13) | (x1 >> 19)
    x1 = x0 ^ x1
    first = True
    for i in range(5):
        rots = _TF_ROTS[i % 2]
        for r in (rots[1:] if first else rots):
            x0 = x0 + x1
            x1 = (x1 << r) | (x1 >> (32 - r))
            x1 = x0 ^ x1
        first = False
        a, c = inj[i]
        if a:
            x0 = x0 + a
        x1 = x1 + c
    return x0 ^ x1


def _gumbel(bits):
    fb = (bits >> 9) | jnp.uint32(0x3F800000)
    f = lax.bitcast_convert_type(fb, jnp.float32) - jnp.float32(1.0)
    u = jnp.maximum(f, _TINY)
    return -jnp.log(-jnp.log(u))


def _lexmax(a, b):
    av, ac = a
    bv, bc = b
    take = (bv > av) | ((bv == av) & (bc < ac))
    return jnp.where(take, bv, av), jnp.where(take, bc, ac)


def _body(x_ref, o_ref, *, T, W, C, CT, NSTR, NFULL):
    b = pl.program_id(0)

    lanes = lax.broadcasted_iota(jnp.int32, (T, W), 1)
    trows = lax.broadcasted_iota(jnp.int32, (T, W), 0)
    pre_c = lanes
    pre_idx = b * CT + trows * C + lanes

    def chunk(no, acc, masked):
        av, ac = acc
        off = no * W
        cv = pre_c + off
        bits = _threefry_bits((pre_idx + off).astype(jnp.uint32))
        g = _gumbel(bits)
        val = x_ref[0, :, pl.ds(no * W, W)] + g
        if masked:
            val = jnp.where(cv < C, val, -jnp.inf)
        upd = val > av
        return jnp.maximum(av, val), jnp.where(upd, cv, ac)

    def step(i, carry):
        return tuple(chunk(i * NSTR + j, carry[j], False)
                     for j in range(NSTR))

    init = tuple((jnp.full((T, W), -jnp.inf, jnp.float32),
                  jnp.zeros((T, W), jnp.int32)) for _ in range(NSTR))
    accs = lax.fori_loop(0, NFULL, step, init)
    accs = list(chunk(NFULL * NSTR + j, accs[j], True) for j in range(NSTR))

    n = NSTR
    while n > 1:
        h = n // 2
        for j in range(h):
            accs[j] = _lexmax(accs[j], accs[j + h])
        n = h
    bv, bc = accs[0]

    n = W
    while n > 1:
        h = n // 2
        bv, bc = _lexmax((bv[:, :h], bc[:, :h]), (bv[:, h:n], bc[:, h:n]))
        n = h
    o_ref[0, :, :] = bc


def _sampler(Xt, *, B, C, T, W, NSTR, NFULL, CPAD, interpret=False):
    return pl.pallas_call(
        functools.partial(_body, T=T, W=W, C=C, CT=C * T, NSTR=NSTR,
                          NFULL=NFULL),
        grid=(B,),
        in_specs=[pl.BlockSpec((1, T, CPAD), lambda b: (b, 0, 0))],
        out_specs=pl.BlockSpec((1, T, 1), lambda b: (b, 0, 0)),
        out_shape=jax.ShapeDtypeStruct((B, T, 1), jnp.int32),
        compiler_params=pltpu.CompilerParams(
            dimension_semantics=("arbitrary",)),
        interpret=interpret,
    )(Xt)


def kernel(X, interpret=False):
    if X.ndim == 2:
        X = X[None]
    B, C, T = X.shape
    assert T == 16, (B, C, T)
    Xt = jnp.transpose(X, (0, 2, 1))
    W = 128
    NSTR = 8
    PER = W * NSTR
    NFULL = C // PER
    if C % PER == 0 and NFULL > 0:
        NFULL -= 1
    CPAD = (NFULL + 1) * PER
    out = _sampler(Xt, B=B, C=C, T=T, W=W, NSTR=NSTR, NFULL=NFULL,
                   CPAD=CPAD, interpret=interpret)
    return out.reshape(B, T)


# speedup vs baseline: 2.6869x; 1.0542x over previous
import functools

import jax
import jax.numpy as jnp
import numpy as np
from jax import lax
from jax.experimental import pallas as pl
from jax.experimental.pallas import tpu as pltpu

_TF_ROTS = ((13, 15, 26, 6), (17, 29, 16, 24))
_TINY = np.float32(1.1754943508222875e-38)


def _threefry_bits(x1):
    k1 = np.uint32(42)
    k2 = np.uint32(42 ^ 0x1BD11BDA)
    ks = (np.uint32(0), k1, k2)
    inj = [(ks[(i + 1) % 3], np.uint32(ks[(i + 2) % 3] + np.uint32(i + 1)))
           for i in range(5)]
    x1 = x1 + k1
    x0 = x1
    x1 = (x1 << 13) | (x1 >> 19)
    x1 = x0 ^ x1
    first = True
    for i in range(5):
        rots = _TF_ROTS[i % 2]
        for r in (rots[1:] if first else rots):
            x0 = x0 + x1
            x1 = (x1 << r) | (x1 >> (32 - r))
            x1 = x0 ^ x1
        first = False
        a, c = inj[i]
        if a:
            x0 = x0 + a
        x1 = x1 + c
    return x0 ^ x1


def _gumbel(bits):
    fb = (bits >> 9) | jnp.uint32(0x3F800000)
    f = lax.bitcast_convert_type(fb, jnp.float32) - jnp.float32(1.0)
    u = jnp.maximum(f, _TINY)
    return -jnp.log(-jnp.log(u))


def _lexmax(a, b):
    av, ac = a
    bv, bc = b
    take = (bv > av) | ((bv == av) & (bc < ac))
    return jnp.where(take, bv, av), jnp.where(take, bc, ac)


def _body(x_ref, o_ref, *, T, W, C, CT, NSTR, NFULL):
    b = pl.program_id(0)

    lanes = lax.broadcasted_iota(jnp.int32, (T, W), 1)
    trows = lax.broadcasted_iota(jnp.int32, (T, W), 0)
    pre_c = lanes
    pre_idx = b * CT + trows * C + lanes

    def chunk(no, acc, masked):
        av, ac = acc
        off = no * W
        cv = pre_c + off
        bits = _threefry_bits((pre_idx + off).astype(jnp.uint32))
        g = _gumbel(bits)
        val = x_ref[0, :, pl.ds(no * W, W)] + g
        if masked:
            val = jnp.where(cv < C, val, -jnp.inf)
        upd = val > av
        return jnp.maximum(av, val), jnp.where(upd, cv, ac)

    def step(i, carry):
        return tuple(chunk(i * NSTR + j, carry[j], False)
                     for j in range(NSTR))

    init = tuple((jnp.full((T, W), -jnp.inf, jnp.float32),
                  jnp.zeros((T, W), jnp.int32)) for _ in range(NSTR))
    accs = lax.fori_loop(0, NFULL, step, init)
    accs = list(chunk(NFULL * NSTR + j, accs[j], True) for j in range(NSTR))

    n = NSTR
    while n > 1:
        h = n // 2
        if n % 2:
            accs[0] = _lexmax(accs[0], accs[n - 1])
        for j in range(h):
            accs[j] = _lexmax(accs[j], accs[j + h])
        n = h
    bv, bc = accs[0]

    n = W
    while n > 1:
        h = n // 2
        bv, bc = _lexmax((bv[:, :h], bc[:, :h]), (bv[:, h:n], bc[:, h:n]))
        n = h
    o_ref[0, :, :] = bc


def _sampler(Xt, *, B, C, T, W, NSTR, NFULL, CPAD, interpret=False):
    return pl.pallas_call(
        functools.partial(_body, T=T, W=W, C=C, CT=C * T, NSTR=NSTR,
                          NFULL=NFULL),
        grid=(B,),
        in_specs=[pl.BlockSpec((1, T, CPAD), lambda b: (b, 0, 0))],
        out_specs=pl.BlockSpec((1, T, 1), lambda b: (b, 0, 0)),
        out_shape=jax.ShapeDtypeStruct((B, T, 1), jnp.int32),
        compiler_params=pltpu.CompilerParams(
            dimension_semantics=("arbitrary",)),
        interpret=interpret,
    )(Xt)


def kernel(X, interpret=False):
    if X.ndim == 2:
        X = X[None]
    B, C, T = X.shape
    assert T == 16, (B, C, T)
    Xt = jnp.transpose(X, (0, 2, 1))
    W = 128
    NSTR = 12
    PER = W * NSTR
    NFULL = C // PER
    if C % PER == 0 and NFULL > 0:
        NFULL -= 1
    CPAD = (NFULL + 1) * PER
    out = _sampler(Xt, B=B, C=C, T=T, W=W, NSTR=NSTR, NFULL=NFULL,
                   CPAD=CPAD, interpret=interpret)
    return out.reshape(B, T)


# speedup vs baseline: 2.7467x; 1.0223x over previous
import functools

import jax
import jax.numpy as jnp
import numpy as np
from jax import lax
from jax.experimental import pallas as pl
from jax.experimental.pallas import tpu as pltpu

_TF_ROTS = ((13, 15, 26, 6), (17, 29, 16, 24))
_TINY = np.float32(1.1754943508222875e-38)


def _threefry_bits(x1):
    k1 = np.uint32(42)
    k2 = np.uint32(42 ^ 0x1BD11BDA)
    ks = (np.uint32(0), k1, k2)
    inj = [(ks[(i + 1) % 3], np.uint32(ks[(i + 2) % 3] + np.uint32(i + 1)))
           for i in range(5)]
    x1 = x1 + k1
    x0 = x1
    x1 = (x1 << 13) | (x1 >> 19)
    x1 = x0 ^ x1
    first = True
    for i in range(5):
        rots = _TF_ROTS[i % 2]
        for r in (rots[1:] if first else rots):
            x0 = x0 + x1
            x1 = (x1 << r) | (x1 >> (32 - r))
            x1 = x0 ^ x1
        first = False
        a, c = inj[i]
        if a:
            x0 = x0 + a
        x1 = x1 + c
    return x0 ^ x1


def _gumbel(bits):
    fb = (bits >> 9) | jnp.uint32(0x3F800000)
    f = lax.bitcast_convert_type(fb, jnp.float32) - jnp.float32(1.0)
    u = jnp.maximum(f, _TINY)
    return -jnp.log(-jnp.log(u))


def _lexmax(a, b):
    av, ac = a
    bv, bc = b
    take = (bv > av) | ((bv == av) & (bc < ac))
    return jnp.where(take, bv, av), jnp.where(take, bc, ac)


def _body(x_ref, o_ref, *, T, W, C, CT, NSTR, NFULL):
    b = pl.program_id(0)

    lanes = lax.broadcasted_iota(jnp.int32, (T, W), 1)
    trows = lax.broadcasted_iota(jnp.int32, (T, W), 0)
    pre_c = lanes
    pre_idx = b * CT + trows * C + lanes

    def chunk(no, acc, masked):
        av, ac = acc
        off = no * W
        cv = pre_c + off
        bits = _threefry_bits((pre_idx + off).astype(jnp.uint32))
        g = _gumbel(bits)
        val = x_ref[0, :, pl.ds(no * W, W)] + g
        if masked:
            val = jnp.where(cv < C, val, -jnp.inf)
        upd = val > av
        return jnp.maximum(av, val), jnp.where(upd, cv, ac)

    def step(i, carry):
        return tuple(chunk(i * NSTR + j, carry[j], False)
                     for j in range(NSTR))

    init = tuple((jnp.full((T, W), -jnp.inf, jnp.float32),
                  jnp.zeros((T, W), jnp.int32)) for _ in range(NSTR))
    accs = lax.fori_loop(0, NFULL, step, init)
    accs = list(chunk(NFULL * NSTR + j, accs[j], True) for j in range(NSTR))

    n = NSTR
    while n > 1:
        h = n // 2
        if n % 2:
            accs[0] = _lexmax(accs[0], accs[n - 1])
        for j in range(h):
            accs[j] = _lexmax(accs[j], accs[j + h])
        n = h
    bv, bc = accs[0]

    n = W
    while n > 1:
        h = n // 2
        bv, bc = _lexmax((bv[:, :h], bc[:, :h]), (bv[:, h:n], bc[:, h:n]))
        n = h
    o_ref[0, :, :] = bc


def _sampler(Xt, *, B, C, T, W, NSTR, NFULL, CPAD, interpret=False):
    return pl.pallas_call(
        functools.partial(_body, T=T, W=W, C=C, CT=C * T, NSTR=NSTR,
                          NFULL=NFULL),
        grid=(B,),
        in_specs=[pl.BlockSpec((1, T, CPAD), lambda b: (b, 0, 0))],
        out_specs=pl.BlockSpec((1, T, 1), lambda b: (b, 0, 0)),
        out_shape=jax.ShapeDtypeStruct((B, T, 1), jnp.int32),
        compiler_params=pltpu.CompilerParams(
            dimension_semantics=("arbitrary",)),
        interpret=interpret,
    )(Xt)


def kernel(X, interpret=False):
    if X.ndim == 2:
        X = X[None]
    B, C, T = X.shape
    assert T == 16, (B, C, T)
    Xt = jnp.transpose(X, (0, 2, 1))
    W = 128
    NSTR = 16
    PER = W * NSTR
    NFULL = C // PER
    if C % PER == 0 and NFULL > 0:
        NFULL -= 1
    CPAD = (NFULL + 1) * PER
    out = _sampler(Xt, B=B, C=C, T=T, W=W, NSTR=NSTR, NFULL=NFULL,
                   CPAD=CPAD, interpret=interpret)
    return out.reshape(B, T)


# speedup vs baseline: 2.7722x; 1.0093x over previous
import functools

import jax
import jax.numpy as jnp
import numpy as np
from jax import lax
from jax.experimental import pallas as pl
from jax.experimental.pallas import tpu as pltpu

_TF_ROTS = ((13, 15, 26, 6), (17, 29, 16, 24))
_TINY = np.float32(1.1754943508222875e-38)


def _threefry_bits(x1):
    k1 = np.uint32(42)
    k2 = np.uint32(42 ^ 0x1BD11BDA)
    ks = (np.uint32(0), k1, k2)
    inj = [(ks[(i + 1) % 3], np.uint32(ks[(i + 2) % 3] + np.uint32(i + 1)))
           for i in range(5)]
    x1 = x1 + k1
    x0 = x1
    x1 = (x1 << 13) | (x1 >> 19)
    x1 = x0 ^ x1
    first = True
    for i in range(5):
        rots = _TF_ROTS[i % 2]
        for r in (rots[1:] if first else rots):
            x0 = x0 + x1
            x1 = (x1 << r) | (x1 >> (32 - r))
            x1 = x0 ^ x1
        first = False
        a, c = inj[i]
        if a:
            x0 = x0 + a
        x1 = x1 + c
    return x0 ^ x1


def _gumbel(bits):
    fb = (bits >> 9) | jnp.uint32(0x3F800000)
    f = lax.bitcast_convert_type(fb, jnp.float32) - jnp.float32(1.0)
    u = jnp.maximum(f, _TINY)
    return -jnp.log(-jnp.log(u))


def _lexmax(a, b):
    av, ac = a
    bv, bc = b
    take = (bv > av) | ((bv == av) & (bc < ac))
    return jnp.where(take, bv, av), jnp.where(take, bc, ac)


def _body(x_ref, o_ref, *, T, W, C, CT, NSTR, NFULL):
    b = pl.program_id(0)

    lanes = lax.broadcasted_iota(jnp.int32, (T, W), 1)
    trows = lax.broadcasted_iota(jnp.int32, (T, W), 0)
    pre_c = lanes
    pre_idx = b * CT + trows * C + lanes

    def chunk(no, acc, masked):
        av, ac = acc
        off = no * W
        cv = pre_c + off
        bits = _threefry_bits((pre_idx + off).astype(jnp.uint32))
        g = _gumbel(bits)
        val = x_ref[0, :, pl.ds(no * W, W)] + g
        if masked:
            val = jnp.where(cv < C, val, -jnp.inf)
        upd = val > av
        return jnp.maximum(av, val), jnp.where(upd, cv, ac)

    def step(i, carry):
        return tuple(chunk(i * NSTR + j, carry[j], False)
                     for j in range(NSTR))

    init = tuple((jnp.full((T, W), -jnp.inf, jnp.float32),
                  jnp.zeros((T, W), jnp.int32)) for _ in range(NSTR))
    accs = lax.fori_loop(0, NFULL, step, init)
    accs = list(chunk(NFULL * NSTR + j, accs[j], True) for j in range(NSTR))

    n = NSTR
    while n > 1:
        h = n // 2
        if n % 2:
            accs[0] = _lexmax(accs[0], accs[n - 1])
        for j in range(h):
            accs[j] = _lexmax(accs[j], accs[j + h])
        n = h
    bv, bc = accs[0]

    n = W
    while n > 1:
        h = n // 2
        bv, bc = _lexmax((bv[:, :h], bc[:, :h]), (bv[:, h:n], bc[:, h:n]))
        n = h
    o_ref[0, :, :] = bc


def _sampler(Xt, *, B, C, T, W, NSTR, NFULL, CPAD, interpret=False):
    return pl.pallas_call(
        functools.partial(_body, T=T, W=W, C=C, CT=C * T, NSTR=NSTR,
                          NFULL=NFULL),
        grid=(B,),
        in_specs=[pl.BlockSpec((1, T, CPAD), lambda b: (b, 0, 0))],
        out_specs=pl.BlockSpec((1, T, 1), lambda b: (b, 0, 0)),
        out_shape=jax.ShapeDtypeStruct((B, T, 1), jnp.int32),
        compiler_params=pltpu.CompilerParams(
            dimension_semantics=("arbitrary",)),
        interpret=interpret,
    )(Xt)


def kernel(X, interpret=False):
    if X.ndim == 2:
        X = X[None]
    B, C, T = X.shape
    assert T == 16, (B, C, T)
    Xt = jnp.transpose(X, (0, 2, 1))
    W = 128
    NSTR = 20
    PER = W * NSTR
    NFULL = C // PER
    if C % PER == 0 and NFULL > 0:
        NFULL -= 1
    CPAD = (NFULL + 1) * PER
    out = _sampler(Xt, B=B, C=C, T=T, W=W, NSTR=NSTR, NFULL=NFULL,
                   CPAD=CPAD, interpret=interpret)
    return out.reshape(B, T)
